# baseline (speedup 1.0000x reference)
import functools

import jax
import jax.numpy as jnp
import numpy as np
from jax import lax
from jax.experimental import pallas as pl
from jax.experimental.pallas import tpu as pltpu
from jax.experimental.pallas import tpu_sc as plsc

B = 16384
F = 26
D = 128
FIELD = 4000
V = F * FIELD

NC = 2
NS = 16
NW = NC * NS
BW = B // NW

CHUNKS = ((2048, 64), (4096, 128), (4096, 128), (4096, 128), (2048, 64))
assert sum(c for c, _ in CHUNKS) == B


def _make_emb_body(per_w, nch, ch):
    def _sc_emb_body(idx_hbm, tab_hbm, emb_out,
                     idx_v, bufA, bufB, semA, semB, semWA, semWB):
        wid = lax.axis_index("s") * NC + lax.axis_index("c")
        base = wid * per_w
        pltpu.sync_copy(idx_hbm.at[wid], idx_v)

        def body(c, carry):
            c0 = c * 2
            c1 = c0 + 1

            @pl.when(c > 0)
            def _drain_prev_writes():
                pltpu.make_async_copy(bufA, emb_out.at[pl.ds(0, ch)], semWA).wait()
                pltpu.make_async_copy(bufB, emb_out.at[pl.ds(0, ch)], semWB).wait()

            gA = pltpu.async_copy(tab_hbm.at[idx_v.at[c0]], bufA, semA)
            gB = pltpu.async_copy(tab_hbm.at[idx_v.at[c1]], bufB, semB)
            gA.wait()
            pltpu.async_copy(bufA, emb_out.at[pl.ds(base + c0 * ch, ch)], semWA)
            gB.wait()
            pltpu.async_copy(bufB, emb_out.at[pl.ds(base + c1 * ch, ch)], semWB)
            return carry

        lax.fori_loop(0, nch // 2, body, 0)
        pltpu.make_async_copy(bufA, emb_out.at[pl.ds(0, ch)], semWA).wait()
        pltpu.make_async_copy(bufB, emb_out.at[pl.ds(0, ch)], semWB).wait()

    return _sc_emb_body


def _sc_wide_body(idx_hbm, lw_hbm, lin_out, idx_v, lw_v, acc_v, sem):
    wid = lax.axis_index("s") * NC + lax.axis_index("c")
    base = wid * BW
    cp_lw = pltpu.async_copy(lw_hbm, lw_v, sem)
    pltpu.sync_copy(idx_hbm.at[wid], idx_v)
    cp_lw.wait()

    def body(j, carry):
        p = j * 16
        acc = jnp.zeros((16,), jnp.float32)
        for f in range(F):
            ivec = idx_v[f, pl.ds(p, 16)]
            acc = acc + plsc.load_gather(lw_v, [ivec])
        acc_v[pl.ds(p, 16)] = acc
        return carry

    lax.fori_loop(0, BW // 16, body, 0)
    pltpu.sync_copy(acc_v, lin_out.at[pl.ds(base, BW)])


@functools.cache
def _mesh():
    return plsc.VectorSubcoreMesh(
        core_axis_name="c", subcore_axis_name="s",
        num_cores=NC, num_subcores=NS)


@functools.cache
def _get_emb_kernel(cb, ch):
    per_w = F * cb // NW
    nch = per_w // ch
    assert nch % 2 == 0 and nch * ch == per_w
    return pl.kernel(
        _make_emb_body(per_w, nch, ch),
        out_type=jax.ShapeDtypeStruct((F * cb, D), jnp.float32),
        mesh=_mesh(),
        scratch_types=[
            pltpu.VMEM((nch, ch), jnp.int32),
            pltpu.VMEM((ch, D), jnp.float32),
            pltpu.VMEM((ch, D), jnp.float32),
            pltpu.SemaphoreType.DMA,
            pltpu.SemaphoreType.DMA,
            pltpu.SemaphoreType.DMA,
            pltpu.SemaphoreType.DMA,
        ],
        compiler_params=pltpu.CompilerParams(has_side_effects=True),
    )


@functools.cache
def _get_wide_kernel():
    return pl.kernel(
        _sc_wide_body,
        out_type=jax.ShapeDtypeStruct((B,), jnp.float32),
        mesh=_mesh(),
        scratch_types=[
            pltpu.VMEM((F, BW), jnp.int32),
            pltpu.VMEM((V,), jnp.float32),
            pltpu.VMEM((BW,), jnp.float32),
            pltpu.SemaphoreType.DMA,
        ],
        compiler_params=pltpu.CompilerParams(
            needs_layout_passes=False, has_side_effects=True),
    )


BB = 512


def _mlp_body(emb_ref, w1_ref, b1_ref, w2_ref, b2_ref, w3_ref,
              b3_ref, wout_ref, out_ref):
    a1 = jnp.zeros((BB, 1024), jnp.float32)
    for f in range(0, F, 2):
        lhs = jnp.concatenate(
            [emb_ref[f].astype(jnp.bfloat16),
             emb_ref[f + 1].astype(jnp.bfloat16)], axis=1)
        rhs = jnp.concatenate([w1_ref[f], w1_ref[f + 1]], axis=0)
        a1 = a1 + jnp.dot(lhs, rhs, preferred_element_type=jnp.float32)
    h1 = jnp.maximum(a1 + b1_ref[...], 0.0).astype(jnp.bfloat16)
    a2 = jnp.dot(h1, w2_ref[...], preferred_element_type=jnp.float32)
    h2 = jnp.maximum(a2 + b2_ref[...], 0.0).astype(jnp.bfloat16)
    a3 = jnp.dot(h2, w3_ref[...], preferred_element_type=jnp.float32)
    h3 = jnp.maximum(a3 + b3_ref[...], 0.0)
    deep = jnp.sum(h3 * wout_ref[...], axis=1)
    out_ref[...] = deep


def _mlp(cb, emb, w1, b1, w2, b2, w3, b3, wout_t):
    grid = (cb // BB,)
    return pl.pallas_call(
        _mlp_body,
        grid=grid,
        in_specs=[
            pl.BlockSpec((F, BB, D), lambda i: (0, i, 0)),
            pl.BlockSpec((F, D, 1024), lambda i: (0, 0, 0)),
            pl.BlockSpec((1, 1024), lambda i: (0, 0)),
            pl.BlockSpec((1024, 512), lambda i: (0, 0)),
            pl.BlockSpec((1, 512), lambda i: (0, 0)),
            pl.BlockSpec((512, 256), lambda i: (0, 0)),
            pl.BlockSpec((1, 256), lambda i: (0, 0)),
            pl.BlockSpec((1, 256), lambda i: (0, 0)),
        ],
        out_specs=pl.BlockSpec((BB,), lambda i: (i,)),
        out_shape=jax.ShapeDtypeStruct((cb,), jnp.float32),
    )(emb, w1, b1, w2, b2, w3, b3, wout_t)


_OFFS = np.arange(F, dtype=np.int32) * FIELD


def kernel(x, linear_w, linear_bias, embed_table, W1, b1, W2, b2, W3, b3,
           Wout, bout):
    idx = x.astype(jnp.int32) + _OFFS[None, :]
    idx_w = idx.reshape(NW, BW, F).transpose(0, 2, 1)

    embs = []
    off = 0
    for cb, ch in CHUNKS:
        per_w = F * cb // NW
        idx_c = idx[off:off + cb].T.reshape(NW, per_w // ch, ch)
        embs.append(_get_emb_kernel(cb, ch)(idx_c, embed_table))
        off += cb
    lin = _get_wide_kernel()(idx_w, linear_w.reshape(V))

    w1r = W1.reshape(F, D, 1024).astype(jnp.bfloat16)
    w2c = W2.astype(jnp.bfloat16)
    w3c = W3.astype(jnp.bfloat16)
    b1r = b1.reshape(1, 1024)
    b2r = b2.reshape(1, 512)
    b3r = b3.reshape(1, 256)
    woutr = Wout.reshape(1, 256)

    outs = []
    for i, (cb, _) in enumerate(CHUNKS):
        outs.append(_mlp(cb, embs[i].reshape(F, cb, D),
                         w1r, b1r, w2c, b2r, w3c, b3r, woutr))
    deep = jnp.concatenate(outs)
    return deep + lin + (bout[0] + linear_bias[0])


# speedup vs baseline: 1.0438x; 1.0438x over previous
import functools

import jax
import jax.numpy as jnp
import numpy as np
from jax import lax
from jax.experimental import pallas as pl
from jax.experimental.pallas import tpu as pltpu
from jax.experimental.pallas import tpu_sc as plsc

B = 16384
F = 26
D = 128
FIELD = 4000
V = F * FIELD

NC = 2
NS = 16
NW = NC * NS
BW = B // NW

CHUNKS = ((4096, 128), (4096, 128), (4096, 128), (4096, 128))
assert sum(c for c, _ in CHUNKS) == B


def _make_emb_body(per_w, nch, ch):
    def _sc_emb_body(idx_hbm, tab_hbm, emb_out,
                     idx_v, bufA, bufB, semA, semB, semWA, semWB):
        wid = lax.axis_index("s") * NC + lax.axis_index("c")
        base = wid * per_w
        pltpu.sync_copy(idx_hbm.at[wid], idx_v)

        def body(c, carry):
            c0 = c * 2
            c1 = c0 + 1

            @pl.when(c > 0)
            def _drain_prev_writes():
                pltpu.make_async_copy(bufA, emb_out.at[pl.ds(0, ch)], semWA).wait()
                pltpu.make_async_copy(bufB, emb_out.at[pl.ds(0, ch)], semWB).wait()

            gA = pltpu.async_copy(tab_hbm.at[idx_v.at[c0]], bufA, semA)
            gB = pltpu.async_copy(tab_hbm.at[idx_v.at[c1]], bufB, semB)
            gA.wait()
            pltpu.async_copy(bufA, emb_out.at[pl.ds(base + c0 * ch, ch)], semWA)
            gB.wait()
            pltpu.async_copy(bufB, emb_out.at[pl.ds(base + c1 * ch, ch)], semWB)
            return carry

        lax.fori_loop(0, nch // 2, body, 0)
        pltpu.make_async_copy(bufA, emb_out.at[pl.ds(0, ch)], semWA).wait()
        pltpu.make_async_copy(bufB, emb_out.at[pl.ds(0, ch)], semWB).wait()

    return _sc_emb_body


def _sc_wide_body(idx_hbm, lw_hbm, lin_out, idx_v, lw_v, acc_v, sem):
    wid = lax.axis_index("s") * NC + lax.axis_index("c")
    base = wid * BW
    cp_lw = pltpu.async_copy(lw_hbm, lw_v, sem)
    pltpu.sync_copy(idx_hbm.at[wid], idx_v)
    cp_lw.wait()

    def body(j, carry):
        p = j * 16
        acc = jnp.zeros((16,), jnp.float32)
        for f in range(F):
            ivec = idx_v[f, pl.ds(p, 16)]
            acc = acc + plsc.load_gather(lw_v, [ivec])
        acc_v[pl.ds(p, 16)] = acc
        return carry

    lax.fori_loop(0, BW // 16, body, 0)
    pltpu.sync_copy(acc_v, lin_out.at[pl.ds(base, BW)])


@functools.cache
def _mesh():
    return plsc.VectorSubcoreMesh(
        core_axis_name="c", subcore_axis_name="s",
        num_cores=NC, num_subcores=NS)


@functools.cache
def _get_emb_kernel(cb, ch):
    per_w = F * cb // NW
    nch = per_w // ch
    assert nch % 2 == 0 and nch * ch == per_w
    return pl.kernel(
        _make_emb_body(per_w, nch, ch),
        out_type=jax.ShapeDtypeStruct((F * cb, D), jnp.float32),
        mesh=_mesh(),
        scratch_types=[
            pltpu.VMEM((nch, ch), jnp.int32),
            pltpu.VMEM((ch, D), jnp.float32),
            pltpu.VMEM((ch, D), jnp.float32),
            pltpu.SemaphoreType.DMA,
            pltpu.SemaphoreType.DMA,
            pltpu.SemaphoreType.DMA,
            pltpu.SemaphoreType.DMA,
        ],
        compiler_params=pltpu.CompilerParams(has_side_effects=True),
    )


@functools.cache
def _get_wide_kernel():
    return pl.kernel(
        _sc_wide_body,
        out_type=jax.ShapeDtypeStruct((B,), jnp.float32),
        mesh=_mesh(),
        scratch_types=[
            pltpu.VMEM((F, BW), jnp.int32),
            pltpu.VMEM((V,), jnp.float32),
            pltpu.VMEM((BW,), jnp.float32),
            pltpu.SemaphoreType.DMA,
        ],
        compiler_params=pltpu.CompilerParams(
            needs_layout_passes=False, has_side_effects=True),
    )


BB = 512


def _mlp_body(emb_ref, w1_ref, b1_ref, w2_ref, b2_ref, w3_ref,
              b3_ref, wout_ref, out_ref):
    a1 = jnp.zeros((BB, 1024), jnp.float32)
    for f in range(0, F, 2):
        lhs = jnp.concatenate(
            [emb_ref[f].astype(jnp.bfloat16),
             emb_ref[f + 1].astype(jnp.bfloat16)], axis=1)
        rhs = jnp.concatenate([w1_ref[f], w1_ref[f + 1]], axis=0)
        a1 = a1 + jnp.dot(lhs, rhs, preferred_element_type=jnp.float32)
    h1 = jnp.maximum(a1 + b1_ref[...], 0.0).astype(jnp.bfloat16)
    a2 = jnp.dot(h1, w2_ref[...], preferred_element_type=jnp.float32)
    h2 = jnp.maximum(a2 + b2_ref[...], 0.0).astype(jnp.bfloat16)
    a3 = jnp.dot(h2, w3_ref[...], preferred_element_type=jnp.float32)
    h3 = jnp.maximum(a3 + b3_ref[...], 0.0)
    deep = jnp.sum(h3 * wout_ref[...], axis=1)
    out_ref[...] = deep


def _mlp(cb, emb, w1, b1, w2, b2, w3, b3, wout_t):
    grid = (cb // BB,)
    return pl.pallas_call(
        _mlp_body,
        grid=grid,
        in_specs=[
            pl.BlockSpec((F, BB, D), lambda i: (0, i, 0)),
            pl.BlockSpec((F, D, 1024), lambda i: (0, 0, 0)),
            pl.BlockSpec((1, 1024), lambda i: (0, 0)),
            pl.BlockSpec((1024, 512), lambda i: (0, 0)),
            pl.BlockSpec((1, 512), lambda i: (0, 0)),
            pl.BlockSpec((512, 256), lambda i: (0, 0)),
            pl.BlockSpec((1, 256), lambda i: (0, 0)),
            pl.BlockSpec((1, 256), lambda i: (0, 0)),
        ],
        out_specs=pl.BlockSpec((BB,), lambda i: (i,)),
        out_shape=jax.ShapeDtypeStruct((cb,), jnp.float32),
    )(emb, w1, b1, w2, b2, w3, b3, wout_t)


_OFFS = np.arange(F, dtype=np.int32) * FIELD


def kernel(x, linear_w, linear_bias, embed_table, W1, b1, W2, b2, W3, b3,
           Wout, bout):
    idx = x.astype(jnp.int32) + _OFFS[None, :]
    idx_w = idx.reshape(NW, BW, F).transpose(0, 2, 1)

    embs = []
    off = 0
    for cb, ch in CHUNKS:
        per_w = F * cb // NW
        idx_c = idx[off:off + cb].T.reshape(NW, per_w // ch, ch)
        embs.append(_get_emb_kernel(cb, ch)(idx_c, embed_table))
        off += cb
    lin = _get_wide_kernel()(idx_w, linear_w.reshape(V))

    w1r = W1.reshape(F, D, 1024).astype(jnp.bfloat16)
    w2c = W2.astype(jnp.bfloat16)
    w3c = W3.astype(jnp.bfloat16)
    b1r = b1.reshape(1, 1024)
    b2r = b2.reshape(1, 512)
    b3r = b3.reshape(1, 256)
    woutr = Wout.reshape(1, 256)

    outs = []
    for i, (cb, _) in enumerate(CHUNKS):
        outs.append(_mlp(cb, embs[i].reshape(F, cb, D),
                         w1r, b1r, w2c, b2r, w3c, b3r, woutr))
    deep = jnp.concatenate(outs)
    return deep + lin + (bout[0] + linear_bias[0])
